# baseline (speedup 1.0000x reference)
import jax
import jax.numpy as jnp
from jax import lax
from jax.experimental import pallas as pl
from jax.experimental.pallas import tpu as pltpu

_PLACEHOLDER = 9
_LR_MUL = 0.1


def _mlp_body(tok_ref, face_ref, w0_ref, b0_ref, w1_ref, b1_ref, cm_ref, cs_ref,
              tie0_ref, tie1_ref, pos_ref):
    x = face_ref[...]
    nrm = jnp.sqrt(jnp.sum(x * x, axis=1, keepdims=True))
    x = x / jnp.maximum(nrm, 1e-12)
    h = lax.dot_general(x, w0_ref[...], (((1,), (1,)), ((), ())),
                        precision=lax.Precision.HIGHEST,
                        preferred_element_type=jnp.float32)
    h = h * _LR_MUL + b0_ref[...] * _LR_MUL
    h = jnp.where(h >= 0, h, 0.2 * h)
    r = lax.dot_general(h, w1_ref[...], (((1,), (1,)), ((), ())),
                        precision=lax.Precision.HIGHEST,
                        preferred_element_type=jnp.float32)
    r = r * _LR_MUL + b1_ref[...] * _LR_MUL
    r = jnp.where(r >= 0, r, 0.2 * r)
    d = cm_ref.shape[1]
    tie0_ref[...] = cm_ref[0:1, :] + r[:, :d] * cs_ref[0:1, :]
    tie1_ref[...] = cm_ref[1:2, :] + r[:, d:] * cs_ref[1:2, :]
    tok = tok_ref[...]
    bm, n_seq = tok.shape
    iota = lax.broadcasted_iota(jnp.int32, (bm, n_seq), 1)
    pos_ref[...] = jnp.min(jnp.where(tok == _PLACEHOLDER, iota, n_seq + 1),
                           axis=1, keepdims=True)


def _merge_body(pos_sref, emb_ref, tie0_ref, tie1_ref, out_ref):
    i = pl.program_id(0)
    bm = out_ref.shape[0]
    out_ref[...] = emb_ref[...]
    for j in range(bm):
        p = pos_sref[i * bm + j]
        out_ref[j, pl.ds(p, 1), :] = tie0_ref[pl.ds(j, 1), :]
        out_ref[j, pl.ds(p + 1, 1), :] = tie1_ref[pl.ds(j, 1), :]


def kernel(tokenized_text, embedded_text, face_img_embeddings,
           W0, b0, W1, b1, celeb_mean, celeb_std):
    batch, n_seq, token_dim = embedded_text.shape
    dim_out = W0.shape[0]
    vit_dim = face_img_embeddings.shape[1]

    bmlp = 256
    tie0, tie1, pos2d = pl.pallas_call(
        _mlp_body,
        grid=(batch // bmlp,),
        in_specs=[
            pl.BlockSpec((bmlp, n_seq), lambda i: (i, 0)),
            pl.BlockSpec((bmlp, vit_dim), lambda i: (i, 0)),
            pl.BlockSpec((dim_out, vit_dim), lambda i: (0, 0)),
            pl.BlockSpec((1, dim_out), lambda i: (0, 0)),
            pl.BlockSpec((dim_out, dim_out), lambda i: (0, 0)),
            pl.BlockSpec((1, dim_out), lambda i: (0, 0)),
            pl.BlockSpec((2, token_dim), lambda i: (0, 0)),
            pl.BlockSpec((2, token_dim), lambda i: (0, 0)),
        ],
        out_specs=(
            pl.BlockSpec((bmlp, token_dim), lambda i: (i, 0)),
            pl.BlockSpec((bmlp, token_dim), lambda i: (i, 0)),
            pl.BlockSpec((bmlp, 1), lambda i: (i, 0)),
        ),
        out_shape=(
            jax.ShapeDtypeStruct((batch, token_dim), jnp.float32),
            jax.ShapeDtypeStruct((batch, token_dim), jnp.float32),
            jax.ShapeDtypeStruct((batch, 1), jnp.int32),
        ),
    )(tokenized_text, face_img_embeddings, W0, b0.reshape(1, dim_out), W1,
      b1.reshape(1, dim_out), celeb_mean, celeb_std)

    pos = pos2d.reshape(batch)
    bm = 32
    out = pl.pallas_call(
        _merge_body,
        grid_spec=pltpu.PrefetchScalarGridSpec(
            num_scalar_prefetch=1,
            grid=(batch // bm,),
            in_specs=[
                pl.BlockSpec((bm, n_seq, token_dim), lambda i, pos_s: (i, 0, 0)),
                pl.BlockSpec((bm, token_dim), lambda i, pos_s: (i, 0)),
                pl.BlockSpec((bm, token_dim), lambda i, pos_s: (i, 0)),
            ],
            out_specs=pl.BlockSpec((bm, n_seq, token_dim),
                                   lambda i, pos_s: (i, 0, 0)),
        ),
        out_shape=jax.ShapeDtypeStruct((batch, n_seq, token_dim), jnp.float32),
    )(pos, embedded_text, tie0, tie1)
    return out
